# baseline (speedup 1.0000x reference)
import jax
import jax.numpy as jnp
from jax import lax
from jax.experimental import pallas as pl

B, N, D = 1024, 128, 128
BC = 32


def _body(rand_ref, row_ref, col_ref):
    r = rand_ref[...]
    a = r[:, :, None]
    b = r[:, None, :]
    k_iota = lax.broadcasted_iota(jnp.int32, (BC, N, N), 2)
    j_iota = lax.broadcasted_iota(jnp.int32, (BC, N, N), 1)
    lt = (b < a) | ((b == a) & (k_iota < j_iota))
    ranks = jnp.sum(lt.astype(jnp.int32), axis=2)
    n_iota = lax.broadcasted_iota(jnp.int32, (BC, N, N), 1)
    col_ref[...] = (ranks[:, None, :] == n_iota).astype(jnp.float32)
    row_ref[...] = jnp.zeros((BC, N, D), jnp.float32)


def kernel(distance_matrix):
    rand = jax.random.uniform(jax.random.key(42), (B, N), dtype=jnp.float32)
    row_emb, col_emb = pl.pallas_call(
        _body,
        grid=(B // BC,),
        in_specs=[pl.BlockSpec((BC, N), lambda i: (i, 0))],
        out_specs=[
            pl.BlockSpec((BC, N, D), lambda i: (i, 0, 0)),
            pl.BlockSpec((BC, N, D), lambda i: (i, 0, 0)),
        ],
        out_shape=[
            jax.ShapeDtypeStruct((B, N, D), jnp.float32),
            jax.ShapeDtypeStruct((B, N, D), jnp.float32),
        ],
    )(rand)
    return (row_emb, col_emb, distance_matrix)


# speedup vs baseline: 18.4557x; 18.4557x over previous
import jax
import jax.numpy as jnp
from jax import lax
from jax.experimental import pallas as pl

B, N, D = 1024, 128, 128
RC = 64
BC = 32


def _rank_body(rand_ref, ranks_ref):
    r = rand_ref[...]
    rj = r[:, None, :]
    rk = r[:, :, None]
    k_iota = lax.broadcasted_iota(jnp.int32, (RC, N, N), 1)
    j_iota = lax.broadcasted_iota(jnp.int32, (RC, N, N), 2)
    lt = (rk < rj) | ((rk == rj) & (k_iota < j_iota))
    ranks_ref[...] = jnp.sum(lt.astype(jnp.int32), axis=1)


def _write_body(ranks_ref, row_ref, col_ref):
    ranks = ranks_ref[...]
    n_iota = lax.broadcasted_iota(jnp.int32, (BC, N, N), 1)
    col_ref[...] = (ranks[:, None, :] == n_iota).astype(jnp.float32)
    row_ref[...] = jnp.zeros((BC, N, D), jnp.float32)


def kernel(distance_matrix):
    rand = jax.random.uniform(jax.random.key(42), (B, N), dtype=jnp.float32)
    ranks = pl.pallas_call(
        _rank_body,
        grid=(B // RC,),
        in_specs=[pl.BlockSpec((RC, N), lambda i: (i, 0))],
        out_specs=pl.BlockSpec((RC, N), lambda i: (i, 0)),
        out_shape=jax.ShapeDtypeStruct((B, N), jnp.int32),
    )(rand)
    row_emb, col_emb = pl.pallas_call(
        _write_body,
        grid=(B // BC,),
        in_specs=[pl.BlockSpec((BC, N), lambda i: (i, 0))],
        out_specs=[
            pl.BlockSpec((BC, N, D), lambda i: (i, 0, 0)),
            pl.BlockSpec((BC, N, D), lambda i: (i, 0, 0)),
        ],
        out_shape=[
            jax.ShapeDtypeStruct((B, N, D), jnp.float32),
            jax.ShapeDtypeStruct((B, N, D), jnp.float32),
        ],
    )(ranks)
    return (row_emb, col_emb, distance_matrix)
